# Initial kernel scaffold; baseline (speedup 1.0000x reference)
import jax
import jax.numpy as jnp
from jax.experimental import pallas as pl


def kernel(x_p, x_h, edge_index_p, edge_index_h, x_p_batch, x_h_batch, label, emb, W_gat, att_src, att_dst, b_gat, Wq, Wk, Wv, W1, b1, W2, b2, Wc1, bc1, Wc2, bc2):
    raise NotImplementedError("write your pallas kernel here")


# speedup vs baseline: 1.3629x; 1.3629x over previous
import math

import jax
import jax.numpy as jnp
from jax.experimental import pallas as pl

D = 256
H = 4
OUT = D // H
NUM_LAYERS = 2
C = 3
B = 16
NEG_SLOPE = 0.2
ROWB = 512
AROWB = 256

_INTERPRET = False


def _full(shape):
    return pl.BlockSpec(shape, lambda i: tuple(0 for _ in shape))


def _rows(shape):
    return pl.BlockSpec(shape, lambda i: (i,) + tuple(0 for _ in shape[1:]))


def _gat_node_body(x_ref, bias_ref, w_ref, a_ref, xh_ref, asd_ref):
    xb = x_ref[...] + bias_ref[...]
    xh = jnp.dot(xb, w_ref[...], preferred_element_type=jnp.float32)
    xh_ref[...] = xh
    asd_ref[...] = jnp.dot(xh, a_ref[...], preferred_element_type=jnp.float32)


def _gat_node(x, bias, W, A_pack):
    n = x.shape[0]
    grid = n // ROWB
    return pl.pallas_call(
        _gat_node_body,
        grid=(grid,),
        in_specs=[_rows((ROWB, D)), _full((1, D)), _full((D, D)), _full((D, 128))],
        out_specs=[_rows((ROWB, D)), _rows((ROWB, 128))],
        out_shape=[
            jax.ShapeDtypeStruct((n, D), jnp.float32),
            jax.ShapeDtypeStruct((n, 128), jnp.float32),
        ],
        interpret=_INTERPRET,
    )(x, bias, W, A_pack)


def _qkv_body(h_ref, p_ref, bias_ref, wq_ref, wk_ref, wv_ref, q_ref, k_ref, v_ref):
    hb = h_ref[...] + bias_ref[...]
    pb = p_ref[...] + bias_ref[...]
    q_ref[...] = jnp.dot(hb, wq_ref[...], preferred_element_type=jnp.float32)
    k_ref[...] = jnp.dot(pb, wk_ref[...], preferred_element_type=jnp.float32)
    v_ref[...] = jnp.dot(pb, wv_ref[...], preferred_element_type=jnp.float32)


def _qkv(h_raw, p_raw, bias, Wq, Wk, Wv):
    n = h_raw.shape[0]
    grid = n // ROWB
    return pl.pallas_call(
        _qkv_body,
        grid=(grid,),
        in_specs=[_rows((ROWB, D)), _rows((ROWB, D)), _full((1, D)),
                  _full((D, D)), _full((D, D)), _full((D, D))],
        out_specs=[_rows((ROWB, D))] * 3,
        out_shape=[jax.ShapeDtypeStruct((n, D), jnp.float32)] * 3,
        interpret=_INTERPRET,
    )(h_raw, p_raw, bias, Wq, Wk, Wv)


def _attn_body(q_ref, k_ref, v_ref, segh_ref, segp_ref, out_ref):
    q = q_ref[...]
    k = k_ref[...]
    v = v_ref[...]
    s = jax.lax.dot_general(q, k, (((1,), (1,)), ((), ())),
                            preferred_element_type=jnp.float32)
    mask = segh_ref[...] == segp_ref[...]
    s = jnp.where(mask, s, -jnp.inf)
    m = jnp.max(s, axis=1, keepdims=True)
    safe_m = jnp.where(m == -jnp.inf, 0.0, m)
    e = jnp.exp((s - safe_m) * (1.0 / math.sqrt(D)))
    denom = jnp.sum(e, axis=1, keepdims=True)
    num = jnp.dot(e, v, preferred_element_type=jnp.float32)
    out_ref[...] = num / jnp.where(denom == 0.0, 1.0, denom)


def _attention(Q, K, V, segh_col, segp_row):
    n = Q.shape[0]
    grid = n // AROWB
    return pl.pallas_call(
        _attn_body,
        grid=(grid,),
        in_specs=[_rows((AROWB, D)), _full((n, D)), _full((n, D)),
                  _rows((AROWB, 1)), _full((1, n))],
        out_specs=_rows((AROWB, D)),
        out_shape=jax.ShapeDtypeStruct((n, D), jnp.float32),
        interpret=_INTERPRET,
    )(Q, K, V, segh_col, segp_row)


def _ffn_body(ph_ref, h_ref, bias_ref, w1_ref, b1_ref, w2_ref, b2_ref, out_ref):
    ph = ph_ref[...]
    hb = h_ref[...] + bias_ref[...]
    w1 = w1_ref[...]
    u = (jnp.dot(ph, w1[0:D], preferred_element_type=jnp.float32)
         + jnp.dot(hb, w1[D:2 * D], preferred_element_type=jnp.float32)
         + jnp.dot(ph - hb, w1[2 * D:3 * D], preferred_element_type=jnp.float32)
         + jnp.dot(ph * hb, w1[3 * D:4 * D], preferred_element_type=jnp.float32)
         + b1_ref[...])
    u = jnp.maximum(u, 0.0)
    out_ref[...] = jnp.dot(u, w2_ref[...], preferred_element_type=jnp.float32) + b2_ref[...]


def _ffn(p_hat, h_raw, bias, W1, b1, W2, b2):
    n = p_hat.shape[0]
    grid = n // ROWB
    return pl.pallas_call(
        _ffn_body,
        grid=(grid,),
        in_specs=[_rows((ROWB, D)), _rows((ROWB, D)), _full((1, D)),
                  _full((4 * D, D)), _full((1, D)), _full((D, D)), _full((1, D))],
        out_specs=_rows((ROWB, D)),
        out_shape=jax.ShapeDtypeStruct((n, D), jnp.float32),
        interpret=_INTERPRET,
    )(p_hat, h_raw, bias, W1, b1, W2, b2)


def _final_body(cmp_ref, v_ref, segh_row_ref, segh_col_ref, segp_row_ref,
                w1_ref, b1_ref, w2_ref, b2_ref,
                wc1_ref, bc1_ref, wc2_ref, bc2_ref, label_ref,
                logits_ref, loss_ref):
    cmp_r = cmp_ref[...]
    v = v_ref[...]
    segh_row = segh_row_ref[...]
    segp_row = segp_row_ref[...]
    n = cmp_r.shape[0]

    bidx = jax.lax.broadcasted_iota(jnp.int32, (B, n), 0)
    mh = (bidx == segh_row).astype(jnp.float32)
    mp = (bidx == segp_row).astype(jnp.float32)

    counts_h = jnp.sum(mh, axis=1, keepdims=True)
    counts_p = jnp.sum(mp, axis=1, keepdims=True)
    len_h = jnp.max(counts_h)
    len_p = jnp.max(counts_p)

    segV = jnp.dot(mp, v, preferred_element_type=jnp.float32)
    php = segV / len_p
    w1 = w1_ref[...]
    u = (jnp.dot(php, w1[0:D] + w1[2 * D:3 * D], preferred_element_type=jnp.float32)
         + b1_ref[...])
    u = jnp.maximum(u, 0.0)
    cmp_pad = jnp.dot(u, w2_ref[...], preferred_element_type=jnp.float32) + b2_ref[...]

    row_sum = jnp.dot(mh, cmp_r, preferred_element_type=jnp.float32)
    segh_col = segh_col_ref[...]
    maxes = []
    for b in range(B):
        mb = jnp.where(segh_col == b, cmp_r, -jnp.inf)
        maxes.append(jnp.max(mb, axis=0, keepdims=True))
    row_max = jnp.concatenate(maxes, axis=0)

    has_pad = counts_h < len_h
    sent_max = jnp.where(has_pad, jnp.maximum(row_max, cmp_pad), row_max)
    n_pad = len_h - counts_h
    sent_mean = (row_sum + n_pad * cmp_pad) / len_h

    wc1 = wc1_ref[...]
    t = (jnp.dot(sent_max, wc1[0:D], preferred_element_type=jnp.float32)
         + jnp.dot(sent_mean, wc1[D:2 * D], preferred_element_type=jnp.float32)
         + bc1_ref[...])
    t = jnp.maximum(t, 0.0)
    logits = jnp.dot(t, wc2_ref[...], preferred_element_type=jnp.float32) + bc2_ref[...]
    logits_ref[...] = logits

    z = label_ref[...]
    lane = jax.lax.broadcasted_iota(jnp.int32, (B, 128), 1)
    term = jnp.maximum(logits, 0.0) - logits * z + jnp.log1p(jnp.exp(-jnp.abs(logits)))
    term = jnp.where(lane < C, term, 0.0)
    loss_ref[...] = jnp.sum(term, keepdims=True).reshape(1, 1) / (B * C)


def _final(cmp_r, V, segh_row, segh_col, segp_row, W1, b1, W2, b2,
           Wc1, bc1, Wc2p, bc2p, label_p):
    n = cmp_r.shape[0]
    return pl.pallas_call(
        _final_body,
        grid=(1,),
        in_specs=[_full((n, D)), _full((n, D)), _full((1, n)), _full((n, 1)),
                  _full((1, n)), _full((4 * D, D)), _full((1, D)), _full((D, D)),
                  _full((1, D)), _full((2 * D, D)), _full((1, D)), _full((D, 128)),
                  _full((1, 128)), _full((B, 128))],
        out_specs=[_full((B, 128)), _full((1, 1))],
        out_shape=[
            jax.ShapeDtypeStruct((B, 128), jnp.float32),
            jax.ShapeDtypeStruct((1, 1), jnp.float32),
        ],
        interpret=_INTERPRET,
    )(cmp_r, V, segh_row, segh_col, segp_row, W1, b1, W2, b2,
      Wc1, bc1, Wc2p, bc2p, label_p)


def _gat_edge(xh, asd, src, dst):
    n = xh.shape[0]
    a_s = asd[:, 0:H]
    a_d = asd[:, H:2 * H]
    alpha = a_s[src] + a_d[dst]
    alpha = jnp.where(alpha > 0, alpha, NEG_SLOPE * alpha)
    amax = jax.ops.segment_max(alpha, dst, num_segments=n)
    ex = jnp.exp(alpha - amax[dst])
    denom = jax.ops.segment_sum(ex, dst, num_segments=n)
    att = ex / (denom[dst] + 1e-16)
    msg = xh[src].reshape(-1, H, OUT) * att[:, :, None]
    out = jax.ops.segment_sum(msg, dst, num_segments=n)
    return out.reshape(n, D)


def _encoder(x, edge_index, bias0, b_gat_row, W_gat, A_pack):
    n = x.shape[0]
    loops = jnp.arange(n, dtype=edge_index.dtype)
    src = jnp.concatenate([edge_index[0], loops])
    dst = jnp.concatenate([edge_index[1], loops])
    bias = bias0
    for _ in range(NUM_LAYERS):
        xh, asd = _gat_node(x, bias, W_gat, A_pack)
        x = _gat_edge(xh, asd, src, dst)
        bias = b_gat_row
    return x


def kernel(x_p, x_h, edge_index_p, edge_index_h, x_p_batch, x_h_batch, label,
           emb, W_gat, att_src, att_dst, b_gat, Wq, Wk, Wv, W1, b1, W2, b2,
           Wc1, bc1, Wc2, bc2):
    n = x_p.shape[0]

    rows = jnp.arange(D)
    head = rows // OUT
    A_pack = jnp.zeros((D, 128), jnp.float32)
    A_pack = A_pack.at[rows, head].set(att_src.reshape(-1))
    A_pack = A_pack.at[rows, head + H].set(att_dst.reshape(-1))
    zero_row = jnp.zeros((1, D), jnp.float32)
    b_gat_row = b_gat.reshape(1, D)
    b1_row = b1.reshape(1, D)
    b2_row = b2.reshape(1, D)
    bc1_row = bc1.reshape(1, D)
    Wc2p = jnp.zeros((D, 128), jnp.float32).at[:, 0:C].set(Wc2)
    bc2p = jnp.zeros((1, 128), jnp.float32).at[0, 0:C].set(bc2)
    label_p = jnp.zeros((B, 128), jnp.float32).at[:, 0:C].set(label.reshape(-1, C))
    segh_row = x_h_batch.reshape(1, n).astype(jnp.int32)
    segh_col = x_h_batch.reshape(n, 1).astype(jnp.int32)
    segp_row = x_p_batch.reshape(1, n).astype(jnp.int32)

    w_p = jnp.take(emb, x_p, axis=0)
    w_h = jnp.take(emb, x_h, axis=0)

    p_raw = _encoder(w_p, edge_index_p, zero_row, b_gat_row, W_gat, A_pack)
    h_raw = _encoder(w_h, edge_index_h, zero_row, b_gat_row, W_gat, A_pack)

    Q, K, V = _qkv(h_raw, p_raw, b_gat_row, Wq, Wk, Wv)
    p_hat = _attention(Q, K, V, segh_col, segp_row)
    cmp_r = _ffn(p_hat, h_raw, b_gat_row, W1, b1_row, W2, b2_row)
    logits_p, loss = _final(cmp_r, V, segh_row, segh_col, segp_row,
                            W1, b1_row, W2, b2_row, Wc1, bc1_row, Wc2p, bc2p,
                            label_p)
    logits = logits_p[:, 0:C]
    return (loss.reshape(()), logits)


# speedup vs baseline: 1.4003x; 1.0274x over previous
import math

import jax
import jax.numpy as jnp
from jax.experimental import pallas as pl

D = 256
H = 4
OUT = D // H
NUM_LAYERS = 2
C = 3
B = 16
NEG_SLOPE = 0.2
ROWB = 512
AROWB = 256

_INTERPRET = False


def _full(shape):
    return pl.BlockSpec(shape, lambda i: tuple(0 for _ in shape))


def _rows(shape):
    return pl.BlockSpec(shape, lambda i: (i,) + tuple(0 for _ in shape[1:]))


def _gat_node_body(x_ref, bias_ref, w_ref, a_ref, xh_ref, asd_ref):
    xb = x_ref[...] + bias_ref[...]
    xh = jnp.dot(xb, w_ref[...], preferred_element_type=jnp.float32)
    xh_ref[...] = xh
    asd_ref[...] = jnp.dot(xh, a_ref[...], preferred_element_type=jnp.float32)


def _gat_node(x, bias, W, A_pack):
    n = x.shape[0]
    grid = n // ROWB
    return pl.pallas_call(
        _gat_node_body,
        grid=(grid,),
        in_specs=[_rows((ROWB, D)), _full((1, D)), _full((D, D)), _full((D, 128))],
        out_specs=[_rows((ROWB, D)), _rows((ROWB, 128))],
        out_shape=[
            jax.ShapeDtypeStruct((n, D), jnp.float32),
            jax.ShapeDtypeStruct((n, 128), jnp.float32),
        ],
        interpret=_INTERPRET,
    )(x, bias, W, A_pack)


def _qkv_body(h_ref, p_ref, bias_ref, wq_ref, wk_ref, wv_ref, q_ref, k_ref, v_ref):
    hb = h_ref[...] + bias_ref[...]
    pb = p_ref[...] + bias_ref[...]
    q_ref[...] = jnp.dot(hb, wq_ref[...], preferred_element_type=jnp.float32)
    k_ref[...] = jnp.dot(pb, wk_ref[...], preferred_element_type=jnp.float32)
    v_ref[...] = jnp.dot(pb, wv_ref[...], preferred_element_type=jnp.float32)


def _qkv(h_raw, p_raw, bias, Wq, Wk, Wv):
    n = h_raw.shape[0]
    grid = n // ROWB
    return pl.pallas_call(
        _qkv_body,
        grid=(grid,),
        in_specs=[_rows((ROWB, D)), _rows((ROWB, D)), _full((1, D)),
                  _full((D, D)), _full((D, D)), _full((D, D))],
        out_specs=[_rows((ROWB, D))] * 3,
        out_shape=[jax.ShapeDtypeStruct((n, D), jnp.float32)] * 3,
        interpret=_INTERPRET,
    )(h_raw, p_raw, bias, Wq, Wk, Wv)


def _attn_body(q_ref, k_ref, v_ref, segh_ref, segp_ref, out_ref):
    q = q_ref[...]
    k = k_ref[...]
    v = v_ref[...]
    s = jax.lax.dot_general(q, k, (((1,), (1,)), ((), ())),
                            preferred_element_type=jnp.float32)
    mask = segh_ref[...] == segp_ref[...]
    s = jnp.where(mask, s, -jnp.inf)
    m = jnp.max(s, axis=1, keepdims=True)
    safe_m = jnp.where(m == -jnp.inf, 0.0, m)
    e = jnp.exp((s - safe_m) * (1.0 / math.sqrt(D)))
    denom = jnp.sum(e, axis=1, keepdims=True)
    num = jnp.dot(e, v, preferred_element_type=jnp.float32)
    out_ref[...] = num / jnp.where(denom == 0.0, 1.0, denom)


def _attention(Q, K, V, segh_col, segp_row):
    n = Q.shape[0]
    grid = n // AROWB
    return pl.pallas_call(
        _attn_body,
        grid=(grid,),
        in_specs=[_rows((AROWB, D)), _full((n, D)), _full((n, D)),
                  _rows((AROWB, 1)), _full((1, n))],
        out_specs=_rows((AROWB, D)),
        out_shape=jax.ShapeDtypeStruct((n, D), jnp.float32),
        interpret=_INTERPRET,
    )(Q, K, V, segh_col, segp_row)


def _ffn_body(ph_ref, h_ref, bias_ref, w1_ref, b1_ref, w2_ref, b2_ref, out_ref):
    ph = ph_ref[...]
    hb = h_ref[...] + bias_ref[...]
    w1 = w1_ref[...]
    u = (jnp.dot(ph, w1[0:D], preferred_element_type=jnp.float32)
         + jnp.dot(hb, w1[D:2 * D], preferred_element_type=jnp.float32)
         + jnp.dot(ph - hb, w1[2 * D:3 * D], preferred_element_type=jnp.float32)
         + jnp.dot(ph * hb, w1[3 * D:4 * D], preferred_element_type=jnp.float32)
         + b1_ref[...])
    u = jnp.maximum(u, 0.0)
    out_ref[...] = jnp.dot(u, w2_ref[...], preferred_element_type=jnp.float32) + b2_ref[...]


def _ffn(p_hat, h_raw, bias, W1, b1, W2, b2):
    n = p_hat.shape[0]
    grid = n // ROWB
    return pl.pallas_call(
        _ffn_body,
        grid=(grid,),
        in_specs=[_rows((ROWB, D)), _rows((ROWB, D)), _full((1, D)),
                  _full((4 * D, D)), _full((1, D)), _full((D, D)), _full((1, D))],
        out_specs=_rows((ROWB, D)),
        out_shape=jax.ShapeDtypeStruct((n, D), jnp.float32),
        interpret=_INTERPRET,
    )(p_hat, h_raw, bias, W1, b1, W2, b2)


def _final_body(cmp_ref, v_ref, segh_row_ref, segh_col_ref, segp_row_ref,
                w1_ref, b1_ref, w2_ref, b2_ref,
                wc1_ref, bc1_ref, wc2_ref, bc2_ref, label_ref,
                logits_ref, loss_ref):
    cmp_r = cmp_ref[...]
    v = v_ref[...]
    segh_row = segh_row_ref[...]
    segp_row = segp_row_ref[...]
    n = cmp_r.shape[0]

    bidx = jax.lax.broadcasted_iota(jnp.int32, (B, n), 0)
    mh = (bidx == segh_row).astype(jnp.float32)
    mp = (bidx == segp_row).astype(jnp.float32)

    counts_h = jnp.sum(mh, axis=1, keepdims=True)
    counts_p = jnp.sum(mp, axis=1, keepdims=True)
    len_h = jnp.max(counts_h)
    len_p = jnp.max(counts_p)

    segV = jnp.dot(mp, v, preferred_element_type=jnp.float32)
    php = segV / len_p
    w1 = w1_ref[...]
    u = (jnp.dot(php, w1[0:D] + w1[2 * D:3 * D], preferred_element_type=jnp.float32)
         + b1_ref[...])
    u = jnp.maximum(u, 0.0)
    cmp_pad = jnp.dot(u, w2_ref[...], preferred_element_type=jnp.float32) + b2_ref[...]

    row_sum = jnp.dot(mh, cmp_r, preferred_element_type=jnp.float32)
    segh_col = segh_col_ref[...]
    maxes = []
    for b in range(B):
        mb = jnp.where(segh_col == b, cmp_r, -jnp.inf)
        maxes.append(jnp.max(mb, axis=0, keepdims=True))
    row_max = jnp.concatenate(maxes, axis=0)

    has_pad = counts_h < len_h
    sent_max = jnp.where(has_pad, jnp.maximum(row_max, cmp_pad), row_max)
    n_pad = len_h - counts_h
    sent_mean = (row_sum + n_pad * cmp_pad) / len_h

    wc1 = wc1_ref[...]
    t = (jnp.dot(sent_max, wc1[0:D], preferred_element_type=jnp.float32)
         + jnp.dot(sent_mean, wc1[D:2 * D], preferred_element_type=jnp.float32)
         + bc1_ref[...])
    t = jnp.maximum(t, 0.0)
    logits = jnp.dot(t, wc2_ref[...], preferred_element_type=jnp.float32) + bc2_ref[...]
    logits_ref[...] = logits

    z = label_ref[...]
    lane = jax.lax.broadcasted_iota(jnp.int32, (B, 128), 1)
    term = jnp.maximum(logits, 0.0) - logits * z + jnp.log1p(jnp.exp(-jnp.abs(logits)))
    term = jnp.where(lane < C, term, 0.0)
    loss_ref[...] = jnp.sum(term, keepdims=True).reshape(1, 1) / (B * C)


def _final(cmp_r, V, segh_row, segh_col, segp_row, W1, b1, W2, b2,
           Wc1, bc1, Wc2p, bc2p, label_p):
    n = cmp_r.shape[0]
    return pl.pallas_call(
        _final_body,
        grid=(1,),
        in_specs=[_full((n, D)), _full((n, D)), _full((1, n)), _full((n, 1)),
                  _full((1, n)), _full((4 * D, D)), _full((1, D)), _full((D, D)),
                  _full((1, D)), _full((2 * D, D)), _full((1, D)), _full((D, 128)),
                  _full((1, 128)), _full((B, 128))],
        out_specs=[_full((B, 128)), _full((1, 1))],
        out_shape=[
            jax.ShapeDtypeStruct((B, 128), jnp.float32),
            jax.ShapeDtypeStruct((1, 1), jnp.float32),
        ],
        interpret=_INTERPRET,
    )(cmp_r, V, segh_row, segh_col, segp_row, W1, b1, W2, b2,
      Wc1, bc1, Wc2p, bc2p, label_p)


def _gat_edge(xh, asd, src, dst):
    n = xh.shape[0]
    a_s = asd[:, 0:H]
    a_d = asd[:, H:2 * H]
    shift = jnp.max(a_s, axis=0, keepdims=True) + a_d
    shift = jnp.where(shift > 0, shift, NEG_SLOPE * shift)
    alpha = a_s[src] + a_d[dst]
    alpha = jnp.where(alpha > 0, alpha, NEG_SLOPE * alpha)
    ex = jnp.exp(alpha - shift[dst])
    denom = jax.ops.segment_sum(ex, dst, num_segments=n)
    att = ex / (denom[dst] + 1e-16)
    msg = xh[src].reshape(-1, H, OUT) * att[:, :, None]
    out = jax.ops.segment_sum(msg, dst, num_segments=n)
    return out.reshape(n, D)


def _encoder(x, src, dst, bias0, b_gat_row, W_gat, A_pack):
    bias = bias0
    for _ in range(NUM_LAYERS):
        xh, asd = _gat_node(x, bias, W_gat, A_pack)
        x = _gat_edge(xh, asd, src, dst)
        bias = b_gat_row
    return x


def kernel(x_p, x_h, edge_index_p, edge_index_h, x_p_batch, x_h_batch, label,
           emb, W_gat, att_src, att_dst, b_gat, Wq, Wk, Wv, W1, b1, W2, b2,
           Wc1, bc1, Wc2, bc2):
    n = x_p.shape[0]

    rows = jnp.arange(D)
    head = rows // OUT
    A_pack = jnp.zeros((D, 128), jnp.float32)
    A_pack = A_pack.at[rows, head].set(att_src.reshape(-1))
    A_pack = A_pack.at[rows, head + H].set(att_dst.reshape(-1))
    zero_row = jnp.zeros((1, D), jnp.float32)
    b_gat_row = b_gat.reshape(1, D)
    b1_row = b1.reshape(1, D)
    b2_row = b2.reshape(1, D)
    bc1_row = bc1.reshape(1, D)
    Wc2p = jnp.zeros((D, 128), jnp.float32).at[:, 0:C].set(Wc2)
    bc2p = jnp.zeros((1, 128), jnp.float32).at[0, 0:C].set(bc2)
    label_p = jnp.zeros((B, 128), jnp.float32).at[:, 0:C].set(label.reshape(-1, C))
    segh_row = x_h_batch.reshape(1, n).astype(jnp.int32)
    segh_col = x_h_batch.reshape(n, 1).astype(jnp.int32)
    segp_row = x_p_batch.reshape(1, n).astype(jnp.int32)

    w_cat = jnp.take(emb, jnp.concatenate([x_p, x_h]), axis=0)
    loops = jnp.arange(2 * n, dtype=edge_index_p.dtype)
    src = jnp.concatenate([edge_index_p[0], edge_index_h[0] + n, loops])
    dst = jnp.concatenate([edge_index_p[1], edge_index_h[1] + n, loops])
    x_enc = _encoder(w_cat, src, dst, zero_row, b_gat_row, W_gat, A_pack)
    p_raw = x_enc[:n]
    h_raw = x_enc[n:]

    Q, K, V = _qkv(h_raw, p_raw, b_gat_row, Wq, Wk, Wv)
    p_hat = _attention(Q, K, V, segh_col, segp_row)
    cmp_r = _ffn(p_hat, h_raw, b_gat_row, W1, b1_row, W2, b2_row)
    logits_p, loss = _final(cmp_r, V, segh_row, segh_col, segp_row,
                            W1, b1_row, W2, b2_row, Wc1, bc1_row, Wc2p, bc2p,
                            label_p)
    logits = logits_p[:, 0:C]
    return (loss.reshape(()), logits)


# speedup vs baseline: 3.7756x; 2.6963x over previous
import math

import jax
import jax.numpy as jnp
from jax.experimental import pallas as pl
from jax.experimental.pallas import tpu as pltpu

D = 256
H = 4
OUT = D // H
NUM_LAYERS = 2
C = 3
B = 16
NEG_SLOPE = 0.2
ROWB = 512
AROWB = 256

_INTERPRET = False


def _full(shape):
    return pl.BlockSpec(shape, lambda i: tuple(0 for _ in shape))


def _rows(shape):
    return pl.BlockSpec(shape, lambda i: (i,) + tuple(0 for _ in shape[1:]))


def _gat_node_body(x_ref, bias_ref, w_ref, a_ref, xh_ref, asd_ref):
    xb = x_ref[...] + bias_ref[...]
    xh = jnp.dot(xb, w_ref[...], preferred_element_type=jnp.float32)
    xh_ref[...] = xh
    asd_ref[...] = jnp.dot(xh, a_ref[...], preferred_element_type=jnp.float32)


def _gat_node(x, bias, W, A_pack):
    n = x.shape[0]
    grid = n // ROWB
    return pl.pallas_call(
        _gat_node_body,
        grid=(grid,),
        in_specs=[_rows((ROWB, D)), _full((1, D)), _full((D, D)), _full((D, 128))],
        out_specs=[_rows((ROWB, D)), _rows((ROWB, 128))],
        out_shape=[
            jax.ShapeDtypeStruct((n, D), jnp.float32),
            jax.ShapeDtypeStruct((n, 128), jnp.float32),
        ],
        interpret=_INTERPRET,
    )(x, bias, W, A_pack)


def _qkv_body(h_ref, p_ref, bias_ref, wq_ref, wk_ref, wv_ref, q_ref, k_ref, v_ref):
    hb = h_ref[...] + bias_ref[...]
    pb = p_ref[...] + bias_ref[...]
    q_ref[...] = jnp.dot(hb, wq_ref[...], preferred_element_type=jnp.float32)
    k_ref[...] = jnp.dot(pb, wk_ref[...], preferred_element_type=jnp.float32)
    v_ref[...] = jnp.dot(pb, wv_ref[...], preferred_element_type=jnp.float32)


def _qkv(h_raw, p_raw, bias, Wq, Wk, Wv):
    n = h_raw.shape[0]
    grid = n // ROWB
    return pl.pallas_call(
        _qkv_body,
        grid=(grid,),
        in_specs=[_rows((ROWB, D)), _rows((ROWB, D)), _full((1, D)),
                  _full((D, D)), _full((D, D)), _full((D, D))],
        out_specs=[_rows((ROWB, D))] * 3,
        out_shape=[jax.ShapeDtypeStruct((n, D), jnp.float32)] * 3,
        interpret=_INTERPRET,
    )(h_raw, p_raw, bias, Wq, Wk, Wv)


def _attn_body(q_ref, k_ref, v_ref, segh_ref, segp_ref, out_ref):
    q = q_ref[...]
    k = k_ref[...]
    v = v_ref[...]
    s = jax.lax.dot_general(q, k, (((1,), (1,)), ((), ())),
                            preferred_element_type=jnp.float32)
    mask = segh_ref[...] == segp_ref[...]
    s = jnp.where(mask, s, -jnp.inf)
    m = jnp.max(s, axis=1, keepdims=True)
    safe_m = jnp.where(m == -jnp.inf, 0.0, m)
    e = jnp.exp((s - safe_m) * (1.0 / math.sqrt(D)))
    denom = jnp.sum(e, axis=1, keepdims=True)
    num = jnp.dot(e, v, preferred_element_type=jnp.float32)
    out_ref[...] = num / jnp.where(denom == 0.0, 1.0, denom)


def _attention(Q, K, V, segh_col, segp_row):
    n = Q.shape[0]
    grid = n // AROWB
    return pl.pallas_call(
        _attn_body,
        grid=(grid,),
        in_specs=[_rows((AROWB, D)), _full((n, D)), _full((n, D)),
                  _rows((AROWB, 1)), _full((1, n))],
        out_specs=_rows((AROWB, D)),
        out_shape=jax.ShapeDtypeStruct((n, D), jnp.float32),
        interpret=_INTERPRET,
    )(Q, K, V, segh_col, segp_row)


def _ffn_body(ph_ref, h_ref, bias_ref, w1_ref, b1_ref, w2_ref, b2_ref, out_ref):
    ph = ph_ref[...]
    hb = h_ref[...] + bias_ref[...]
    w1 = w1_ref[...]
    u = (jnp.dot(ph, w1[0:D], preferred_element_type=jnp.float32)
         + jnp.dot(hb, w1[D:2 * D], preferred_element_type=jnp.float32)
         + jnp.dot(ph - hb, w1[2 * D:3 * D], preferred_element_type=jnp.float32)
         + jnp.dot(ph * hb, w1[3 * D:4 * D], preferred_element_type=jnp.float32)
         + b1_ref[...])
    u = jnp.maximum(u, 0.0)
    out_ref[...] = jnp.dot(u, w2_ref[...], preferred_element_type=jnp.float32) + b2_ref[...]


def _ffn(p_hat, h_raw, bias, W1, b1, W2, b2):
    n = p_hat.shape[0]
    grid = n // ROWB
    return pl.pallas_call(
        _ffn_body,
        grid=(grid,),
        in_specs=[_rows((ROWB, D)), _rows((ROWB, D)), _full((1, D)),
                  _full((4 * D, D)), _full((1, D)), _full((D, D)), _full((1, D))],
        out_specs=_rows((ROWB, D)),
        out_shape=jax.ShapeDtypeStruct((n, D), jnp.float32),
        interpret=_INTERPRET,
    )(p_hat, h_raw, bias, W1, b1, W2, b2)


def _final_body(cmp_ref, v_ref, segh_row_ref, segh_col_ref, segp_row_ref,
                w1_ref, b1_ref, w2_ref, b2_ref,
                wc1_ref, bc1_ref, wc2_ref, bc2_ref, label_ref,
                logits_ref, loss_ref):
    cmp_r = cmp_ref[...]
    v = v_ref[...]
    segh_row = segh_row_ref[...]
    segp_row = segp_row_ref[...]
    n = cmp_r.shape[0]

    bidx = jax.lax.broadcasted_iota(jnp.int32, (B, n), 0)
    mh = (bidx == segh_row).astype(jnp.float32)
    mp = (bidx == segp_row).astype(jnp.float32)

    counts_h = jnp.sum(mh, axis=1, keepdims=True)
    counts_p = jnp.sum(mp, axis=1, keepdims=True)
    len_h = jnp.max(counts_h)
    len_p = jnp.max(counts_p)

    segV = jnp.dot(mp, v, preferred_element_type=jnp.float32)
    php = segV / len_p
    w1 = w1_ref[...]
    u = (jnp.dot(php, w1[0:D] + w1[2 * D:3 * D], preferred_element_type=jnp.float32)
         + b1_ref[...])
    u = jnp.maximum(u, 0.0)
    cmp_pad = jnp.dot(u, w2_ref[...], preferred_element_type=jnp.float32) + b2_ref[...]

    row_sum = jnp.dot(mh, cmp_r, preferred_element_type=jnp.float32)
    segh_col = segh_col_ref[...]
    maxes = []
    for b in range(B):
        mb = jnp.where(segh_col == b, cmp_r, -jnp.inf)
        maxes.append(jnp.max(mb, axis=0, keepdims=True))
    row_max = jnp.concatenate(maxes, axis=0)

    has_pad = counts_h < len_h
    sent_max = jnp.where(has_pad, jnp.maximum(row_max, cmp_pad), row_max)
    n_pad = len_h - counts_h
    sent_mean = (row_sum + n_pad * cmp_pad) / len_h

    wc1 = wc1_ref[...]
    t = (jnp.dot(sent_max, wc1[0:D], preferred_element_type=jnp.float32)
         + jnp.dot(sent_mean, wc1[D:2 * D], preferred_element_type=jnp.float32)
         + bc1_ref[...])
    t = jnp.maximum(t, 0.0)
    logits = jnp.dot(t, wc2_ref[...], preferred_element_type=jnp.float32) + bc2_ref[...]
    logits_ref[...] = logits

    z = label_ref[...]
    lane = jax.lax.broadcasted_iota(jnp.int32, (B, 128), 1)
    term = jnp.maximum(logits, 0.0) - logits * z + jnp.log1p(jnp.exp(-jnp.abs(logits)))
    term = jnp.where(lane < C, term, 0.0)
    loss_ref[...] = jnp.sum(term, keepdims=True).reshape(1, 1) / (B * C)


def _final(cmp_r, V, segh_row, segh_col, segp_row, W1, b1, W2, b2,
           Wc1, bc1, Wc2p, bc2p, label_p):
    n = cmp_r.shape[0]
    return pl.pallas_call(
        _final_body,
        grid=(1,),
        in_specs=[_full((n, D)), _full((n, D)), _full((1, n)), _full((n, 1)),
                  _full((1, n)), _full((4 * D, D)), _full((1, D)), _full((D, D)),
                  _full((1, D)), _full((2 * D, D)), _full((1, D)), _full((D, 128)),
                  _full((1, 128)), _full((B, 128))],
        out_specs=[_full((B, 128)), _full((1, 1))],
        out_shape=[
            jax.ShapeDtypeStruct((B, 128), jnp.float32),
            jax.ShapeDtypeStruct((1, 1), jnp.float32),
        ],
        interpret=_INTERPRET,
    )(cmp_r, V, segh_row, segh_col, segp_row, W1, b1, W2, b2,
      Wc1, bc1, Wc2p, bc2p, label_p)


ETILE = 512


def _edge_body(xh_ref, ps_ref, pd_ref, psh_ref, hsel_ref,
               srcc_ref, dstc_ref, dstr_ref, out_ref,
               denom_ref, exs_ref):
    p = pl.program_id(0)
    t = pl.program_id(1)
    nt = pl.num_programs(1)
    n = xh_ref.shape[0]

    src_col = srcc_ref[...].reshape(ETILE, 1)
    dst_col = dstc_ref[...].reshape(ETILE, 1)
    dst_row = dstr_ref[...].reshape(1, ETILE)
    lane_e = jax.lax.broadcasted_iota(jnp.int32, (ETILE, n), 1)
    oh_src = (lane_e == src_col).astype(jnp.bfloat16)

    @pl.when(p == 0)
    def _phase0():
        oh_dst = (lane_e == dst_col).astype(jnp.bfloat16)
        a = (jnp.dot(oh_src, ps_ref[...], preferred_element_type=jnp.float32)
             + jnp.dot(oh_dst, pd_ref[...], preferred_element_type=jnp.float32))
        sh = jnp.dot(oh_dst, psh_ref[...], preferred_element_type=jnp.float32)
        a = jnp.where(a > 0, a, NEG_SLOPE * a)
        lane = jax.lax.broadcasted_iota(jnp.int32, (ETILE, 128), 1)
        ex = jnp.where(lane < H, jnp.exp(a - sh), 0.0)
        exs_ref[pl.ds(t * ETILE, ETILE), :] = ex.astype(jnp.bfloat16)
        oh_dst_t = (jax.lax.broadcasted_iota(jnp.int32, (n, ETILE), 0)
                    == dst_row).astype(jnp.bfloat16)
        contrib = jnp.dot(oh_dst_t, ex.astype(jnp.bfloat16),
                          preferred_element_type=jnp.float32)
        denom_ref[...] = jnp.where(t == 0, contrib, denom_ref[...] + contrib)

    @pl.when(p == 1)
    def _phase1():
        ex = exs_ref[pl.ds(t * ETILE, ETILE), :]
        ex_full = jnp.dot(ex, hsel_ref[...],
                          preferred_element_type=jnp.float32)
        g = jnp.dot(oh_src, xh_ref[...],
                    preferred_element_type=jnp.float32)
        msg = (g * ex_full).astype(jnp.bfloat16)
        oh_dst_t = (jax.lax.broadcasted_iota(jnp.int32, (n, ETILE), 0)
                    == dst_row).astype(jnp.bfloat16)
        contrib = jnp.dot(oh_dst_t, msg, preferred_element_type=jnp.float32)
        acc = jnp.where(t == 0, contrib, out_ref[...] + contrib)
        out_ref[...] = acc

        @pl.when(t == nt - 1)
        def _finish():
            dfull = jnp.dot(denom_ref[...].astype(jnp.bfloat16), hsel_ref[...],
                            preferred_element_type=jnp.float32)
            out_ref[...] = out_ref[...] / (dfull + 1e-16)


def _gat_edge_pallas(xh_bf, pack_s, pack_d, pack_sh, hsel, src_col, dst_col,
                     dst_row):
    n = xh_bf.shape[0]
    ntiles = src_col.shape[0]
    return pl.pallas_call(
        _edge_body,
        grid=(2, ntiles),
        in_specs=[
            _full2((n, D)), _full2((n, 128)), _full2((n, 128)),
            _full2((n, 128)), _full2((128, D)),
            pl.BlockSpec((1, ETILE, 1), lambda p, t: (t, 0, 0)),
            pl.BlockSpec((1, ETILE, 1), lambda p, t: (t, 0, 0)),
            pl.BlockSpec((1, 1, ETILE), lambda p, t: (t, 0, 0)),
        ],
        out_specs=_full2((n, D)),
        out_shape=jax.ShapeDtypeStruct((n, D), jnp.float32),
        scratch_shapes=[
            pltpu.VMEM((n, 128), jnp.float32),
            pltpu.VMEM((ntiles * ETILE, 128), jnp.bfloat16),
        ],
        interpret=_INTERPRET,
    )(xh_bf, pack_s, pack_d, pack_sh, hsel, src_col, dst_col, dst_row)


def _full2(shape):
    return pl.BlockSpec(shape, lambda p, t: tuple(0 for _ in shape))


def _gat_edge(xh, asd, src_col, dst_col, dst_row, hsel, lane4):
    n = xh.shape[0]
    a_s = asd[:, 0:H]
    a_d = asd[:, H:2 * H]
    shift = jnp.max(a_s, axis=0, keepdims=True) + a_d
    shift = jnp.where(shift > 0, shift, NEG_SLOPE * shift)
    pack_s = jnp.where(lane4, asd, 0.0).astype(jnp.bfloat16)
    pack_d = jnp.where(lane4, jnp.roll(asd, -H, axis=1), 0.0).astype(jnp.bfloat16)
    pack_sh = jnp.zeros((n, 128), jnp.float32).at[:, 0:H].set(shift).astype(jnp.bfloat16)
    xh_bf = xh.astype(jnp.bfloat16)
    return _gat_edge_pallas(xh_bf, pack_s, pack_d, pack_sh, hsel,
                            src_col, dst_col, dst_row)


def _encoder(x, src_col, dst_col, dst_row, hsel, lane4, bias0, b_gat_row,
             W_gat, A_pack):
    bias = bias0
    for _ in range(NUM_LAYERS):
        xh, asd = _gat_node(x, bias, W_gat, A_pack)
        x = _gat_edge(xh, asd, src_col, dst_col, dst_row, hsel, lane4)
        bias = b_gat_row
    return x


def kernel(x_p, x_h, edge_index_p, edge_index_h, x_p_batch, x_h_batch, label,
           emb, W_gat, att_src, att_dst, b_gat, Wq, Wk, Wv, W1, b1, W2, b2,
           Wc1, bc1, Wc2, bc2):
    n = x_p.shape[0]

    rows = jnp.arange(D)
    head = rows // OUT
    A_pack = jnp.zeros((D, 128), jnp.float32)
    A_pack = A_pack.at[rows, head].set(att_src.reshape(-1))
    A_pack = A_pack.at[rows, head + H].set(att_dst.reshape(-1))
    zero_row = jnp.zeros((1, D), jnp.float32)
    b_gat_row = b_gat.reshape(1, D)
    b1_row = b1.reshape(1, D)
    b2_row = b2.reshape(1, D)
    bc1_row = bc1.reshape(1, D)
    Wc2p = jnp.zeros((D, 128), jnp.float32).at[:, 0:C].set(Wc2)
    bc2p = jnp.zeros((1, 128), jnp.float32).at[0, 0:C].set(bc2)
    label_p = jnp.zeros((B, 128), jnp.float32).at[:, 0:C].set(label.reshape(-1, C))
    segh_row = x_h_batch.reshape(1, n).astype(jnp.int32)
    segh_col = x_h_batch.reshape(n, 1).astype(jnp.int32)
    segp_row = x_p_batch.reshape(1, n).astype(jnp.int32)

    w_cat = jnp.take(emb, jnp.concatenate([x_p, x_h]), axis=0)
    loops = jnp.arange(2 * n, dtype=jnp.int32)
    src = jnp.concatenate([edge_index_p[0].astype(jnp.int32),
                           edge_index_h[0].astype(jnp.int32) + n, loops])
    dst = jnp.concatenate([edge_index_p[1].astype(jnp.int32),
                           edge_index_h[1].astype(jnp.int32) + n, loops])
    ntiles = src.shape[0] // ETILE
    src_col = src.reshape(ntiles, ETILE, 1)
    dst_col = dst.reshape(ntiles, ETILE, 1)
    dst_row = dst.reshape(ntiles, 1, ETILE)
    lane128 = jnp.arange(128)[None, :]
    lane4 = lane128 < H
    hsel = (jnp.arange(D)[None, :] // OUT == jnp.arange(128)[:, None]
            ).astype(jnp.bfloat16)
    x_enc = _encoder(w_cat, src_col, dst_col, dst_row, hsel, lane4,
                     zero_row, b_gat_row, W_gat, A_pack)
    p_raw = x_enc[:n]
    h_raw = x_enc[n:]

    Q, K, V = _qkv(h_raw, p_raw, b_gat_row, Wq, Wk, Wv)
    p_hat = _attention(Q, K, V, segh_col, segp_row)
    cmp_r = _ffn(p_hat, h_raw, b_gat_row, W1, b1_row, W2, b2_row)
    logits_p, loss = _final(cmp_r, V, segh_row, segh_col, segp_row,
                            W1, b1_row, W2, b2_row, Wc1, bc1_row, Wc2p, bc2p,
                            label_p)
    logits = logits_p[:, 0:C]
    return (loss.reshape(()), logits)


# speedup vs baseline: 4.6073x; 1.2203x over previous
import math

import jax
import jax.numpy as jnp
from jax.experimental import pallas as pl
from jax.experimental.pallas import tpu as pltpu

D = 256
H = 4
OUT = D // H
NUM_LAYERS = 2
C = 3
B = 16
NEG_SLOPE = 0.2
ROWB = 512
AROWB = 256

_INTERPRET = False


def _full(shape):
    return pl.BlockSpec(shape, lambda i: tuple(0 for _ in shape))


def _rows(shape):
    return pl.BlockSpec(shape, lambda i: (i,) + tuple(0 for _ in shape[1:]))


def _gat_node_body(x_ref, bias_ref, w_ref, a_ref, xh_ref, asd_ref):
    xb = x_ref[...] + bias_ref[...]
    xh = jnp.dot(xb, w_ref[...], preferred_element_type=jnp.float32)
    xh_ref[...] = xh
    asd_ref[...] = jnp.dot(xh, a_ref[...], preferred_element_type=jnp.float32)


def _gat_node(x, bias, W, A_pack):
    n = x.shape[0]
    grid = n // ROWB
    return pl.pallas_call(
        _gat_node_body,
        grid=(grid,),
        in_specs=[_rows((ROWB, D)), _full((1, D)), _full((D, D)), _full((D, 128))],
        out_specs=[_rows((ROWB, D)), _rows((ROWB, 128))],
        out_shape=[
            jax.ShapeDtypeStruct((n, D), jnp.float32),
            jax.ShapeDtypeStruct((n, 128), jnp.float32),
        ],
        interpret=_INTERPRET,
    )(x, bias, W, A_pack)


def _qkv_body(h_ref, p_ref, bias_ref, wq_ref, wk_ref, wv_ref, q_ref, k_ref, v_ref):
    hb = h_ref[...] + bias_ref[...]
    pb = p_ref[...] + bias_ref[...]
    q_ref[...] = jnp.dot(hb, wq_ref[...], preferred_element_type=jnp.float32)
    k_ref[...] = jnp.dot(pb, wk_ref[...], preferred_element_type=jnp.float32)
    v_ref[...] = jnp.dot(pb, wv_ref[...], preferred_element_type=jnp.float32)


def _qkv(h_raw, p_raw, bias, Wq, Wk, Wv):
    n = h_raw.shape[0]
    grid = n // ROWB
    return pl.pallas_call(
        _qkv_body,
        grid=(grid,),
        in_specs=[_rows((ROWB, D)), _rows((ROWB, D)), _full((1, D)),
                  _full((D, D)), _full((D, D)), _full((D, D))],
        out_specs=[_rows((ROWB, D))] * 3,
        out_shape=[jax.ShapeDtypeStruct((n, D), jnp.float32)] * 3,
        interpret=_INTERPRET,
    )(h_raw, p_raw, bias, Wq, Wk, Wv)


def _attn_body(q_ref, k_ref, v_ref, segh_ref, segp_ref, out_ref):
    q = q_ref[...]
    k = k_ref[...]
    v = v_ref[...]
    s = jax.lax.dot_general(q, k, (((1,), (1,)), ((), ())),
                            preferred_element_type=jnp.float32)
    mask = segh_ref[...] == segp_ref[...]
    s = jnp.where(mask, s, -jnp.inf)
    m = jnp.max(s, axis=1, keepdims=True)
    safe_m = jnp.where(m == -jnp.inf, 0.0, m)
    e = jnp.exp((s - safe_m) * (1.0 / math.sqrt(D)))
    denom = jnp.sum(e, axis=1, keepdims=True)
    num = jnp.dot(e, v, preferred_element_type=jnp.float32)
    out_ref[...] = num / jnp.where(denom == 0.0, 1.0, denom)


def _attention(Q, K, V, segh_col, segp_row):
    n = Q.shape[0]
    grid = n // AROWB
    return pl.pallas_call(
        _attn_body,
        grid=(grid,),
        in_specs=[_rows((AROWB, D)), _full((n, D)), _full((n, D)),
                  _rows((AROWB, 1)), _full((1, n))],
        out_specs=_rows((AROWB, D)),
        out_shape=jax.ShapeDtypeStruct((n, D), jnp.float32),
        interpret=_INTERPRET,
    )(Q, K, V, segh_col, segp_row)


def _ffn_body(ph_ref, h_ref, bias_ref, w1_ref, b1_ref, w2_ref, b2_ref, out_ref):
    ph = ph_ref[...]
    hb = h_ref[...] + bias_ref[...]
    w1 = w1_ref[...]
    u = (jnp.dot(ph, w1[0:D], preferred_element_type=jnp.float32)
         + jnp.dot(hb, w1[D:2 * D], preferred_element_type=jnp.float32)
         + jnp.dot(ph - hb, w1[2 * D:3 * D], preferred_element_type=jnp.float32)
         + jnp.dot(ph * hb, w1[3 * D:4 * D], preferred_element_type=jnp.float32)
         + b1_ref[...])
    u = jnp.maximum(u, 0.0)
    out_ref[...] = jnp.dot(u, w2_ref[...], preferred_element_type=jnp.float32) + b2_ref[...]


def _ffn(p_hat, h_raw, bias, W1, b1, W2, b2):
    n = p_hat.shape[0]
    grid = n // ROWB
    return pl.pallas_call(
        _ffn_body,
        grid=(grid,),
        in_specs=[_rows((ROWB, D)), _rows((ROWB, D)), _full((1, D)),
                  _full((4 * D, D)), _full((1, D)), _full((D, D)), _full((1, D))],
        out_specs=_rows((ROWB, D)),
        out_shape=jax.ShapeDtypeStruct((n, D), jnp.float32),
        interpret=_INTERPRET,
    )(p_hat, h_raw, bias, W1, b1, W2, b2)


def _final_body(cmp_ref, v_ref, segh_row_ref, segh_col_ref, segp_row_ref,
                w1_ref, b1_ref, w2_ref, b2_ref,
                wc1_ref, bc1_ref, wc2_ref, bc2_ref, label_ref,
                logits_ref, loss_ref):
    cmp_r = cmp_ref[...]
    v = v_ref[...]
    segh_row = segh_row_ref[...]
    segp_row = segp_row_ref[...]
    n = cmp_r.shape[0]

    bidx = jax.lax.broadcasted_iota(jnp.int32, (B, n), 0)
    mh = (bidx == segh_row).astype(jnp.float32)
    mp = (bidx == segp_row).astype(jnp.float32)

    counts_h = jnp.sum(mh, axis=1, keepdims=True)
    counts_p = jnp.sum(mp, axis=1, keepdims=True)
    len_h = jnp.max(counts_h)
    len_p = jnp.max(counts_p)

    segV = jnp.dot(mp, v, preferred_element_type=jnp.float32)
    php = segV / len_p
    w1 = w1_ref[...]
    u = (jnp.dot(php, w1[0:D] + w1[2 * D:3 * D], preferred_element_type=jnp.float32)
         + b1_ref[...])
    u = jnp.maximum(u, 0.0)
    cmp_pad = jnp.dot(u, w2_ref[...], preferred_element_type=jnp.float32) + b2_ref[...]

    row_sum = jnp.dot(mh, cmp_r, preferred_element_type=jnp.float32)
    segh_col = segh_col_ref[...]
    maxes = []
    for b in range(B):
        mb = jnp.where(segh_col == b, cmp_r, -jnp.inf)
        maxes.append(jnp.max(mb, axis=0, keepdims=True))
    row_max = jnp.concatenate(maxes, axis=0)

    has_pad = counts_h < len_h
    sent_max = jnp.where(has_pad, jnp.maximum(row_max, cmp_pad), row_max)
    n_pad = len_h - counts_h
    sent_mean = (row_sum + n_pad * cmp_pad) / len_h

    wc1 = wc1_ref[...]
    t = (jnp.dot(sent_max, wc1[0:D], preferred_element_type=jnp.float32)
         + jnp.dot(sent_mean, wc1[D:2 * D], preferred_element_type=jnp.float32)
         + bc1_ref[...])
    t = jnp.maximum(t, 0.0)
    logits = jnp.dot(t, wc2_ref[...], preferred_element_type=jnp.float32) + bc2_ref[...]
    logits_ref[...] = logits

    z = label_ref[...]
    lane = jax.lax.broadcasted_iota(jnp.int32, (B, 128), 1)
    term = jnp.maximum(logits, 0.0) - logits * z + jnp.log1p(jnp.exp(-jnp.abs(logits)))
    term = jnp.where(lane < C, term, 0.0)
    loss_ref[...] = jnp.sum(term, keepdims=True).reshape(1, 1) / (B * C)


def _final(cmp_r, V, segh_row, segh_col, segp_row, W1, b1, W2, b2,
           Wc1, bc1, Wc2p, bc2p, label_p):
    n = cmp_r.shape[0]
    return pl.pallas_call(
        _final_body,
        grid=(1,),
        in_specs=[_full((n, D)), _full((n, D)), _full((1, n)), _full((n, 1)),
                  _full((1, n)), _full((4 * D, D)), _full((1, D)), _full((D, D)),
                  _full((1, D)), _full((2 * D, D)), _full((1, D)), _full((D, 128)),
                  _full((1, 128)), _full((B, 128))],
        out_specs=[_full((B, 128)), _full((1, 1))],
        out_shape=[
            jax.ShapeDtypeStruct((B, 128), jnp.float32),
            jax.ShapeDtypeStruct((1, 1), jnp.float32),
        ],
        interpret=_INTERPRET,
    )(cmp_r, V, segh_row, segh_col, segp_row, W1, b1, W2, b2,
      Wc1, bc1, Wc2p, bc2p, label_p)


ETILE = 512


def _edge_body(xh_ref, ps_ref, pd_ref, psh_ref, hsel_ref,
               srcc_ref, dstc_ref, dstr_ref, out_ref, acc_ref):
    t = pl.program_id(0)
    nt = pl.num_programs(0)
    n = xh_ref.shape[0]

    src_col = srcc_ref[...].reshape(ETILE, 1)
    dst_col = dstc_ref[...].reshape(ETILE, 1)
    dst_row = dstr_ref[...].reshape(1, ETILE)
    lane_e = jax.lax.broadcasted_iota(jnp.int32, (ETILE, n), 1)
    oh_src = (lane_e == src_col).astype(jnp.bfloat16)
    oh_dst = (lane_e == dst_col).astype(jnp.bfloat16)

    a = (jnp.dot(oh_src, ps_ref[...], preferred_element_type=jnp.float32)
         + jnp.dot(oh_dst, pd_ref[...], preferred_element_type=jnp.float32))
    sh = jnp.dot(oh_dst, psh_ref[...], preferred_element_type=jnp.float32)
    a = jnp.where(a > 0, a, NEG_SLOPE * a)
    lane = jax.lax.broadcasted_iota(jnp.int32, (ETILE, 128), 1)
    ex = jnp.where(lane < H, jnp.exp(a - sh), 0.0)
    ex_full = jnp.dot(ex.astype(jnp.bfloat16), hsel_ref[...],
                      preferred_element_type=jnp.float32)
    g = jnp.dot(oh_src, xh_ref[...],
                preferred_element_type=jnp.float32)
    msg = (g * ex_full).astype(jnp.bfloat16)
    pay = jnp.concatenate([msg, ex.astype(jnp.bfloat16)], axis=1)
    oh_dst_t = (jax.lax.broadcasted_iota(jnp.int32, (n, ETILE), 0)
                == dst_row).astype(jnp.bfloat16)
    contrib = jnp.dot(oh_dst_t, pay, preferred_element_type=jnp.float32)
    acc = jnp.where(t == 0, contrib, acc_ref[...] + contrib)
    acc_ref[...] = acc

    @pl.when(t == nt - 1)
    def _finish():
        dfull = jnp.dot(acc[:, D:].astype(jnp.bfloat16), hsel_ref[...],
                        preferred_element_type=jnp.float32)
        out_ref[...] = acc[:, :D] / (dfull + 1e-16)


def _gat_edge_pallas(xh_bf, pack_s, pack_d, pack_sh, hsel, src_col, dst_col,
                     dst_row):
    n = xh_bf.shape[0]
    ntiles = src_col.shape[0]
    return pl.pallas_call(
        _edge_body,
        grid=(ntiles,),
        in_specs=[
            _full((n, D)), _full((n, 128)), _full((n, 128)),
            _full((n, 128)), _full((128, D)),
            pl.BlockSpec((1, ETILE, 1), lambda t: (t, 0, 0)),
            pl.BlockSpec((1, ETILE, 1), lambda t: (t, 0, 0)),
            pl.BlockSpec((1, 1, ETILE), lambda t: (t, 0, 0)),
        ],
        out_specs=_full((n, D)),
        out_shape=jax.ShapeDtypeStruct((n, D), jnp.float32),
        scratch_shapes=[
            pltpu.VMEM((n, D + 128), jnp.float32),
        ],
        interpret=_INTERPRET,
    )(xh_bf, pack_s, pack_d, pack_sh, hsel, src_col, dst_col, dst_row)


def _gat_edge(xh, asd, src_col, dst_col, dst_row, hsel, lane4):
    n = xh.shape[0]
    a_s = asd[:, 0:H]
    a_d = asd[:, H:2 * H]
    shift = jnp.max(a_s, axis=0, keepdims=True) + a_d
    shift = jnp.where(shift > 0, shift, NEG_SLOPE * shift)
    pack_s = jnp.where(lane4, asd, 0.0).astype(jnp.bfloat16)
    pack_d = jnp.where(lane4, jnp.roll(asd, -H, axis=1), 0.0).astype(jnp.bfloat16)
    pack_sh = jnp.zeros((n, 128), jnp.float32).at[:, 0:H].set(shift).astype(jnp.bfloat16)
    xh_bf = xh.astype(jnp.bfloat16)
    return _gat_edge_pallas(xh_bf, pack_s, pack_d, pack_sh, hsel,
                            src_col, dst_col, dst_row)


def _encoder(x, src_col, dst_col, dst_row, hsel, lane4, bias0, b_gat_row,
             W_gat, A_pack):
    bias = bias0
    for _ in range(NUM_LAYERS):
        xh, asd = _gat_node(x, bias, W_gat, A_pack)
        x = _gat_edge(xh, asd, src_col, dst_col, dst_row, hsel, lane4)
        bias = b_gat_row
    return x


def kernel(x_p, x_h, edge_index_p, edge_index_h, x_p_batch, x_h_batch, label,
           emb, W_gat, att_src, att_dst, b_gat, Wq, Wk, Wv, W1, b1, W2, b2,
           Wc1, bc1, Wc2, bc2):
    n = x_p.shape[0]

    rows = jnp.arange(D)
    head = rows // OUT
    A_pack = jnp.zeros((D, 128), jnp.float32)
    A_pack = A_pack.at[rows, head].set(att_src.reshape(-1))
    A_pack = A_pack.at[rows, head + H].set(att_dst.reshape(-1))
    zero_row = jnp.zeros((1, D), jnp.float32)
    b_gat_row = b_gat.reshape(1, D)
    b1_row = b1.reshape(1, D)
    b2_row = b2.reshape(1, D)
    bc1_row = bc1.reshape(1, D)
    Wc2p = jnp.zeros((D, 128), jnp.float32).at[:, 0:C].set(Wc2)
    bc2p = jnp.zeros((1, 128), jnp.float32).at[0, 0:C].set(bc2)
    label_p = jnp.zeros((B, 128), jnp.float32).at[:, 0:C].set(label.reshape(-1, C))
    segh_row = x_h_batch.reshape(1, n).astype(jnp.int32)
    segh_col = x_h_batch.reshape(n, 1).astype(jnp.int32)
    segp_row = x_p_batch.reshape(1, n).astype(jnp.int32)

    w_cat = jnp.take(emb, jnp.concatenate([x_p, x_h]), axis=0)
    loops = jnp.arange(2 * n, dtype=jnp.int32)
    src = jnp.concatenate([edge_index_p[0].astype(jnp.int32),
                           edge_index_h[0].astype(jnp.int32) + n, loops])
    dst = jnp.concatenate([edge_index_p[1].astype(jnp.int32),
                           edge_index_h[1].astype(jnp.int32) + n, loops])
    ntiles = src.shape[0] // ETILE
    src_col = src.reshape(ntiles, ETILE, 1)
    dst_col = dst.reshape(ntiles, ETILE, 1)
    dst_row = dst.reshape(ntiles, 1, ETILE)
    lane128 = jnp.arange(128)[None, :]
    lane4 = lane128 < H
    hsel = (jnp.arange(D)[None, :] // OUT == jnp.arange(128)[:, None]
            ).astype(jnp.bfloat16)
    x_enc = _encoder(w_cat, src_col, dst_col, dst_row, hsel, lane4,
                     zero_row, b_gat_row, W_gat, A_pack)
    p_raw = x_enc[:n]
    h_raw = x_enc[n:]

    Q, K, V = _qkv(h_raw, p_raw, b_gat_row, Wq, Wk, Wv)
    p_hat = _attention(Q, K, V, segh_col, segp_row)
    cmp_r = _ffn(p_hat, h_raw, b_gat_row, W1, b1_row, W2, b2_row)
    logits_p, loss = _final(cmp_r, V, segh_row, segh_col, segp_row,
                            W1, b1_row, W2, b2_row, Wc1, bc1_row, Wc2p, bc2p,
                            label_p)
    logits = logits_p[:, 0:C]
    return (loss.reshape(()), logits)


# speedup vs baseline: 6.0011x; 1.3025x over previous
import math

import jax
import jax.numpy as jnp
from jax.experimental import pallas as pl
from jax.experimental.pallas import tpu as pltpu

D = 256
H = 4
OUT = D // H
NUM_LAYERS = 2
C = 3
B = 16
NEG_SLOPE = 0.2
ROWB = 512
AROWB = 256

_INTERPRET = False


def _full(shape):
    return pl.BlockSpec(shape, lambda i: tuple(0 for _ in shape))


def _rows(shape):
    return pl.BlockSpec(shape, lambda i: (i,) + tuple(0 for _ in shape[1:]))


def _gat_node_body(x_ref, bias_ref, w_ref, a_ref, xh_ref, asd_ref):
    xb = x_ref[...] + bias_ref[...]
    xh = jnp.dot(xb, w_ref[...], preferred_element_type=jnp.float32)
    xh_ref[...] = xh
    asd_ref[...] = jnp.dot(xh, a_ref[...], preferred_element_type=jnp.float32)


def _gat_node(x, bias, W, A_pack):
    n = x.shape[0]
    grid = n // ROWB
    return pl.pallas_call(
        _gat_node_body,
        grid=(grid,),
        in_specs=[_rows((ROWB, D)), _full((1, D)), _full((D, D)), _full((D, 128))],
        out_specs=[_rows((ROWB, D)), _rows((ROWB, 128))],
        out_shape=[
            jax.ShapeDtypeStruct((n, D), jnp.float32),
            jax.ShapeDtypeStruct((n, 128), jnp.float32),
        ],
        interpret=_INTERPRET,
    )(x, bias, W, A_pack)


def _qkv_body(h_ref, p_ref, bias_ref, wq_ref, wk_ref, wv_ref, q_ref, k_ref, v_ref):
    hb = h_ref[...] + bias_ref[...]
    pb = p_ref[...] + bias_ref[...]
    q_ref[...] = jnp.dot(hb, wq_ref[...], preferred_element_type=jnp.float32)
    k_ref[...] = jnp.dot(pb, wk_ref[...], preferred_element_type=jnp.float32)
    v_ref[...] = jnp.dot(pb, wv_ref[...], preferred_element_type=jnp.float32)


def _qkv(h_raw, p_raw, bias, Wq, Wk, Wv):
    n = h_raw.shape[0]
    grid = n // ROWB
    return pl.pallas_call(
        _qkv_body,
        grid=(grid,),
        in_specs=[_rows((ROWB, D)), _rows((ROWB, D)), _full((1, D)),
                  _full((D, D)), _full((D, D)), _full((D, D))],
        out_specs=[_rows((ROWB, D))] * 3,
        out_shape=[jax.ShapeDtypeStruct((n, D), jnp.float32)] * 3,
        interpret=_INTERPRET,
    )(h_raw, p_raw, bias, Wq, Wk, Wv)


def _attn_body(q_ref, k_ref, v_ref, segh_ref, segp_ref, out_ref):
    q = q_ref[...]
    k = k_ref[...]
    v = v_ref[...]
    s = jax.lax.dot_general(q, k, (((1,), (1,)), ((), ())),
                            preferred_element_type=jnp.float32)
    mask = segh_ref[...] == segp_ref[...]
    s = jnp.where(mask, s, -jnp.inf)
    m = jnp.max(s, axis=1, keepdims=True)
    safe_m = jnp.where(m == -jnp.inf, 0.0, m)
    e = jnp.exp((s - safe_m) * (1.0 / math.sqrt(D)))
    denom = jnp.sum(e, axis=1, keepdims=True)
    num = jnp.dot(e, v, preferred_element_type=jnp.float32)
    out_ref[...] = num / jnp.where(denom == 0.0, 1.0, denom)


def _attention(Q, K, V, segh_col, segp_row):
    n = Q.shape[0]
    grid = n // AROWB
    return pl.pallas_call(
        _attn_body,
        grid=(grid,),
        in_specs=[_rows((AROWB, D)), _full((n, D)), _full((n, D)),
                  _rows((AROWB, 1)), _full((1, n))],
        out_specs=_rows((AROWB, D)),
        out_shape=jax.ShapeDtypeStruct((n, D), jnp.float32),
        interpret=_INTERPRET,
    )(Q, K, V, segh_col, segp_row)


def _ffn_body(ph_ref, h_ref, bias_ref, w1_ref, b1_ref, w2_ref, b2_ref, out_ref):
    ph = ph_ref[...]
    hb = h_ref[...] + bias_ref[...]
    w1 = w1_ref[...]
    u = (jnp.dot(ph, w1[0:D], preferred_element_type=jnp.float32)
         + jnp.dot(hb, w1[D:2 * D], preferred_element_type=jnp.float32)
         + jnp.dot(ph - hb, w1[2 * D:3 * D], preferred_element_type=jnp.float32)
         + jnp.dot(ph * hb, w1[3 * D:4 * D], preferred_element_type=jnp.float32)
         + b1_ref[...])
    u = jnp.maximum(u, 0.0)
    out_ref[...] = jnp.dot(u, w2_ref[...], preferred_element_type=jnp.float32) + b2_ref[...]


def _ffn(p_hat, h_raw, bias, W1, b1, W2, b2):
    n = p_hat.shape[0]
    grid = n // ROWB
    return pl.pallas_call(
        _ffn_body,
        grid=(grid,),
        in_specs=[_rows((ROWB, D)), _rows((ROWB, D)), _full((1, D)),
                  _full((4 * D, D)), _full((1, D)), _full((D, D)), _full((1, D))],
        out_specs=_rows((ROWB, D)),
        out_shape=jax.ShapeDtypeStruct((n, D), jnp.float32),
        interpret=_INTERPRET,
    )(p_hat, h_raw, bias, W1, b1, W2, b2)


def _final_body(cmp_ref, v_ref, segh_row_ref, segh_col_ref, segp_row_ref,
                w1_ref, b1_ref, w2_ref, b2_ref,
                wc1_ref, bc1_ref, wc2_ref, bc2_ref, label_ref,
                logits_ref, loss_ref):
    cmp_r = cmp_ref[...]
    v = v_ref[...]
    segh_row = segh_row_ref[...]
    segp_row = segp_row_ref[...]
    n = cmp_r.shape[0]

    bidx = jax.lax.broadcasted_iota(jnp.int32, (B, n), 0)
    mh = (bidx == segh_row).astype(jnp.float32)
    mp = (bidx == segp_row).astype(jnp.float32)

    counts_h = jnp.sum(mh, axis=1, keepdims=True)
    counts_p = jnp.sum(mp, axis=1, keepdims=True)
    len_h = jnp.max(counts_h)
    len_p = jnp.max(counts_p)

    segV = jnp.dot(mp, v, preferred_element_type=jnp.float32)
    php = segV / len_p
    w1 = w1_ref[...]
    u = (jnp.dot(php, w1[0:D] + w1[2 * D:3 * D], preferred_element_type=jnp.float32)
         + b1_ref[...])
    u = jnp.maximum(u, 0.0)
    cmp_pad = jnp.dot(u, w2_ref[...], preferred_element_type=jnp.float32) + b2_ref[...]

    row_sum = jnp.dot(mh, cmp_r, preferred_element_type=jnp.float32)
    segh_col = segh_col_ref[...]
    maxes = []
    for b in range(B):
        mb = jnp.where(segh_col == b, cmp_r, -jnp.inf)
        maxes.append(jnp.max(mb, axis=0, keepdims=True))
    row_max = jnp.concatenate(maxes, axis=0)

    has_pad = counts_h < len_h
    sent_max = jnp.where(has_pad, jnp.maximum(row_max, cmp_pad), row_max)
    n_pad = len_h - counts_h
    sent_mean = (row_sum + n_pad * cmp_pad) / len_h

    wc1 = wc1_ref[...]
    t = (jnp.dot(sent_max, wc1[0:D], preferred_element_type=jnp.float32)
         + jnp.dot(sent_mean, wc1[D:2 * D], preferred_element_type=jnp.float32)
         + bc1_ref[...])
    t = jnp.maximum(t, 0.0)
    logits = jnp.dot(t, wc2_ref[...], preferred_element_type=jnp.float32) + bc2_ref[...]
    logits_ref[...] = logits

    z = label_ref[...]
    lane = jax.lax.broadcasted_iota(jnp.int32, (B, 128), 1)
    term = jnp.maximum(logits, 0.0) - logits * z + jnp.log1p(jnp.exp(-jnp.abs(logits)))
    term = jnp.where(lane < C, term, 0.0)
    loss_ref[...] = jnp.sum(term, keepdims=True).reshape(1, 1) / (B * C)


def _final(cmp_r, V, segh_row, segh_col, segp_row, W1, b1, W2, b2,
           Wc1, bc1, Wc2p, bc2p, label_p):
    n = cmp_r.shape[0]
    return pl.pallas_call(
        _final_body,
        grid=(1,),
        in_specs=[_full((n, D)), _full((n, D)), _full((1, n)), _full((n, 1)),
                  _full((1, n)), _full((4 * D, D)), _full((1, D)), _full((D, D)),
                  _full((1, D)), _full((2 * D, D)), _full((1, D)), _full((D, 128)),
                  _full((1, 128)), _full((B, 128))],
        out_specs=[_full((B, 128)), _full((1, 1))],
        out_shape=[
            jax.ShapeDtypeStruct((B, 128), jnp.float32),
            jax.ShapeDtypeStruct((1, 1), jnp.float32),
        ],
        interpret=_INTERPRET,
    )(cmp_r, V, segh_row, segh_col, segp_row, W1, b1, W2, b2,
      Wc1, bc1, Wc2p, bc2p, label_p)


ETILE = 512


def _edge_body(xh_ref, ps_ref, pd_ref, shr_ref, hsel_ref,
               srcc_ref, dstc_ref, dstr_ref, out_ref, acc_ref):
    t = pl.program_id(0)
    nt = pl.num_programs(0)
    n = xh_ref.shape[0]

    src_col = srcc_ref[...].reshape(ETILE, 1)
    dst_col = dstc_ref[...].reshape(ETILE, 1)
    dst_row = dstr_ref[...].reshape(1, ETILE)
    lane_e = jax.lax.broadcasted_iota(jnp.int32, (ETILE, n), 1)
    oh_src = (lane_e == src_col).astype(jnp.bfloat16)
    oh_dst = (lane_e == dst_col).astype(jnp.bfloat16)

    a = (jnp.dot(oh_src, ps_ref[...], preferred_element_type=jnp.float32)
         + jnp.dot(oh_dst, pd_ref[...], preferred_element_type=jnp.float32))
    a = jnp.where(a > 0, a, NEG_SLOPE * a)
    lane = jax.lax.broadcasted_iota(jnp.int32, (ETILE, 128), 1)
    ex = jnp.where(lane < H, jnp.exp(a - shr_ref[...]), 0.0)
    ex_full = jnp.dot(ex.astype(jnp.bfloat16), hsel_ref[...],
                      preferred_element_type=jnp.float32)
    g = jnp.dot(oh_src, xh_ref[...],
                preferred_element_type=jnp.float32)
    msg = (g * ex_full).astype(jnp.bfloat16)
    pay = jnp.concatenate([msg, ex.astype(jnp.bfloat16)], axis=1)
    oh_dst_t = (jax.lax.broadcasted_iota(jnp.int32, (n, ETILE), 0)
                == dst_row).astype(jnp.bfloat16)
    contrib = jnp.dot(oh_dst_t, pay, preferred_element_type=jnp.float32)
    acc = jnp.where(t == 0, contrib, acc_ref[...] + contrib)
    acc_ref[...] = acc

    @pl.when(t == nt - 1)
    def _finish():
        lane_n = jax.lax.broadcasted_iota(jnp.int32, (n, 128), 1)
        a_self = ps_ref[...].astype(jnp.float32) + pd_ref[...].astype(jnp.float32)
        a_self = jnp.where(a_self > 0, a_self, NEG_SLOPE * a_self)
        ex_self = jnp.where(lane_n < H, jnp.exp(a_self - shr_ref[...]), 0.0)
        exs_full = jnp.dot(ex_self.astype(jnp.bfloat16), hsel_ref[...],
                           preferred_element_type=jnp.float32)
        m_self = exs_full * xh_ref[...].astype(jnp.float32)
        dfull = jnp.dot((acc[:, D:] + ex_self).astype(jnp.bfloat16),
                        hsel_ref[...],
                        preferred_element_type=jnp.float32)
        out_ref[...] = (acc[:, :D] + m_self) / (dfull + 1e-16)


def _gat_edge_pallas(xh_bf, pack_s, pack_d, shift_row, hsel, src_col, dst_col,
                     dst_row):
    n = xh_bf.shape[0]
    ntiles = src_col.shape[0]
    return pl.pallas_call(
        _edge_body,
        grid=(ntiles,),
        in_specs=[
            _full((n, D)), _full((n, 128)), _full((n, 128)),
            _full((1, 128)), _full((128, D)),
            pl.BlockSpec((1, ETILE, 1), lambda t: (t, 0, 0)),
            pl.BlockSpec((1, ETILE, 1), lambda t: (t, 0, 0)),
            pl.BlockSpec((1, 1, ETILE), lambda t: (t, 0, 0)),
        ],
        out_specs=_full((n, D)),
        out_shape=jax.ShapeDtypeStruct((n, D), jnp.float32),
        scratch_shapes=[
            pltpu.VMEM((n, D + 128), jnp.float32),
        ],
        interpret=_INTERPRET,
    )(xh_bf, pack_s, pack_d, shift_row, hsel, src_col, dst_col, dst_row)


def _gat_edge(xh, asd, src_col, dst_col, dst_row, hsel, lane4):
    n = xh.shape[0]
    a_max = jnp.max(asd[:, 0:2 * H], axis=0)
    c = a_max[0:H] + a_max[H:2 * H]
    c = jnp.where(c > 0, c, NEG_SLOPE * c)
    shift_row = jnp.zeros((1, 128), jnp.float32).at[0, 0:H].set(c)
    pack_s = jnp.where(lane4, asd, 0.0).astype(jnp.bfloat16)
    pack_d = jnp.where(lane4, jnp.roll(asd, -H, axis=1), 0.0).astype(jnp.bfloat16)
    xh_bf = xh.astype(jnp.bfloat16)
    return _gat_edge_pallas(xh_bf, pack_s, pack_d, shift_row, hsel,
                            src_col, dst_col, dst_row)


def _encoder(x, src_col, dst_col, dst_row, hsel, lane4, bias0, b_gat_row,
             W_gat, A_pack):
    bias = bias0
    for _ in range(NUM_LAYERS):
        xh, asd = _gat_node(x, bias, W_gat, A_pack)
        x = _gat_edge(xh, asd, src_col, dst_col, dst_row, hsel, lane4)
        bias = b_gat_row
    return x


def kernel(x_p, x_h, edge_index_p, edge_index_h, x_p_batch, x_h_batch, label,
           emb, W_gat, att_src, att_dst, b_gat, Wq, Wk, Wv, W1, b1, W2, b2,
           Wc1, bc1, Wc2, bc2):
    n = x_p.shape[0]

    rows = jnp.arange(D)
    head = rows // OUT
    A_pack = jnp.zeros((D, 128), jnp.float32)
    A_pack = A_pack.at[rows, head].set(att_src.reshape(-1))
    A_pack = A_pack.at[rows, head + H].set(att_dst.reshape(-1))
    zero_row = jnp.zeros((1, D), jnp.float32)
    b_gat_row = b_gat.reshape(1, D)
    b1_row = b1.reshape(1, D)
    b2_row = b2.reshape(1, D)
    bc1_row = bc1.reshape(1, D)
    Wc2p = jnp.zeros((D, 128), jnp.float32).at[:, 0:C].set(Wc2)
    bc2p = jnp.zeros((1, 128), jnp.float32).at[0, 0:C].set(bc2)
    label_p = jnp.zeros((B, 128), jnp.float32).at[:, 0:C].set(label.reshape(-1, C))
    segh_row = x_h_batch.reshape(1, n).astype(jnp.int32)
    segh_col = x_h_batch.reshape(n, 1).astype(jnp.int32)
    segp_row = x_p_batch.reshape(1, n).astype(jnp.int32)

    w_cat = jnp.take(emb, jnp.concatenate([x_p, x_h]), axis=0)
    src = jnp.concatenate([edge_index_p[0].astype(jnp.int32),
                           edge_index_h[0].astype(jnp.int32) + n])
    dst = jnp.concatenate([edge_index_p[1].astype(jnp.int32),
                           edge_index_h[1].astype(jnp.int32) + n])
    ntiles = src.shape[0] // ETILE
    src_col = src.reshape(ntiles, ETILE, 1)
    dst_col = dst.reshape(ntiles, ETILE, 1)
    dst_row = dst.reshape(ntiles, 1, ETILE)
    lane128 = jnp.arange(128)[None, :]
    lane4 = lane128 < H
    hsel = (jnp.arange(D)[None, :] // OUT == jnp.arange(128)[:, None]
            ).astype(jnp.bfloat16)
    x_enc = _encoder(w_cat, src_col, dst_col, dst_row, hsel, lane4,
                     zero_row, b_gat_row, W_gat, A_pack)
    p_raw = x_enc[:n]
    h_raw = x_enc[n:]

    Q, K, V = _qkv(h_raw, p_raw, b_gat_row, Wq, Wk, Wv)
    p_hat = _attention(Q, K, V, segh_col, segp_row)
    cmp_r = _ffn(p_hat, h_raw, b_gat_row, W1, b1_row, W2, b2_row)
    logits_p, loss = _final(cmp_r, V, segh_row, segh_col, segp_row,
                            W1, b1_row, W2, b2_row, Wc1, bc1_row, Wc2p, bc2p,
                            label_p)
    logits = logits_p[:, 0:C]
    return (loss.reshape(()), logits)


# speedup vs baseline: 6.0015x; 1.0001x over previous
import math

import jax
import jax.numpy as jnp
from jax.experimental import pallas as pl
from jax.experimental.pallas import tpu as pltpu

D = 256
H = 4
OUT = D // H
NUM_LAYERS = 2
C = 3
B = 16
NEG_SLOPE = 0.2
ROWB = 512
AROWB = 256


def _full(shape):
    return pl.BlockSpec(shape, lambda i: tuple(0 for _ in shape))


def _rows(shape):
    return pl.BlockSpec(shape, lambda i: (i,) + tuple(0 for _ in shape[1:]))


def _gat_node_body(x_ref, bias_ref, w_ref, a_ref, xh_ref, asd_ref):
    xb = x_ref[...] + bias_ref[...]
    xh = jnp.dot(xb, w_ref[...], preferred_element_type=jnp.float32)
    xh_ref[...] = xh
    asd_ref[...] = jnp.dot(xh, a_ref[...], preferred_element_type=jnp.float32)


def _gat_node(x, bias, W, A_pack):
    n = x.shape[0]
    grid = n // ROWB
    return pl.pallas_call(
        _gat_node_body,
        grid=(grid,),
        in_specs=[_rows((ROWB, D)), _full((1, D)), _full((D, D)), _full((D, 128))],
        out_specs=[_rows((ROWB, D)), _rows((ROWB, 128))],
        out_shape=[
            jax.ShapeDtypeStruct((n, D), jnp.float32),
            jax.ShapeDtypeStruct((n, 128), jnp.float32),
        ],
    )(x, bias, W, A_pack)


def _qkv_body(h_ref, p_ref, bias_ref, wq_ref, wk_ref, wv_ref, q_ref, k_ref, v_ref):
    hb = h_ref[...] + bias_ref[...]
    pb = p_ref[...] + bias_ref[...]
    q_ref[...] = jnp.dot(hb, wq_ref[...], preferred_element_type=jnp.float32)
    k_ref[...] = jnp.dot(pb, wk_ref[...], preferred_element_type=jnp.float32)
    v_ref[...] = jnp.dot(pb, wv_ref[...], preferred_element_type=jnp.float32)


def _qkv(h_raw, p_raw, bias, Wq, Wk, Wv):
    n = h_raw.shape[0]
    grid = n // ROWB
    return pl.pallas_call(
        _qkv_body,
        grid=(grid,),
        in_specs=[_rows((ROWB, D)), _rows((ROWB, D)), _full((1, D)),
                  _full((D, D)), _full((D, D)), _full((D, D))],
        out_specs=[_rows((ROWB, D))] * 3,
        out_shape=[jax.ShapeDtypeStruct((n, D), jnp.float32)] * 3,
    )(h_raw, p_raw, bias, Wq, Wk, Wv)


def _attn_body(q_ref, k_ref, v_ref, segh_ref, segp_ref, out_ref):
    q = q_ref[...]
    k = k_ref[...]
    v = v_ref[...]
    s = jax.lax.dot_general(q, k, (((1,), (1,)), ((), ())),
                            preferred_element_type=jnp.float32)
    mask = segh_ref[...] == segp_ref[...]
    s = jnp.where(mask, s, -jnp.inf)
    m = jnp.max(s, axis=1, keepdims=True)
    safe_m = jnp.where(m == -jnp.inf, 0.0, m)
    e = jnp.exp((s - safe_m) * (1.0 / math.sqrt(D)))
    denom = jnp.sum(e, axis=1, keepdims=True)
    num = jnp.dot(e, v, preferred_element_type=jnp.float32)
    out_ref[...] = num / jnp.where(denom == 0.0, 1.0, denom)


def _attention(Q, K, V, segh_col, segp_row):
    n = Q.shape[0]
    grid = n // AROWB
    return pl.pallas_call(
        _attn_body,
        grid=(grid,),
        in_specs=[_rows((AROWB, D)), _full((n, D)), _full((n, D)),
                  _rows((AROWB, 1)), _full((1, n))],
        out_specs=_rows((AROWB, D)),
        out_shape=jax.ShapeDtypeStruct((n, D), jnp.float32),
    )(Q, K, V, segh_col, segp_row)


def _ffn_body(ph_ref, h_ref, bias_ref, w1_ref, b1_ref, w2_ref, b2_ref, out_ref):
    ph = ph_ref[...]
    hb = h_ref[...] + bias_ref[...]
    w1 = w1_ref[...]
    u = (jnp.dot(ph, w1[0:D], preferred_element_type=jnp.float32)
         + jnp.dot(hb, w1[D:2 * D], preferred_element_type=jnp.float32)
         + jnp.dot(ph - hb, w1[2 * D:3 * D], preferred_element_type=jnp.float32)
         + jnp.dot(ph * hb, w1[3 * D:4 * D], preferred_element_type=jnp.float32)
         + b1_ref[...])
    u = jnp.maximum(u, 0.0)
    out_ref[...] = jnp.dot(u, w2_ref[...], preferred_element_type=jnp.float32) + b2_ref[...]


def _ffn(p_hat, h_raw, bias, W1, b1, W2, b2):
    n = p_hat.shape[0]
    grid = n // ROWB
    return pl.pallas_call(
        _ffn_body,
        grid=(grid,),
        in_specs=[_rows((ROWB, D)), _rows((ROWB, D)), _full((1, D)),
                  _full((4 * D, D)), _full((1, D)), _full((D, D)), _full((1, D))],
        out_specs=_rows((ROWB, D)),
        out_shape=jax.ShapeDtypeStruct((n, D), jnp.float32),
    )(p_hat, h_raw, bias, W1, b1, W2, b2)


def _final_body(cmp_ref, v_ref, segh_row_ref, segh_col_ref, segp_row_ref,
                w1_ref, b1_ref, w2_ref, b2_ref,
                wc1_ref, bc1_ref, wc2_ref, bc2_ref, label_ref,
                logits_ref, loss_ref):
    cmp_r = cmp_ref[...]
    v = v_ref[...]
    segh_row = segh_row_ref[...]
    segp_row = segp_row_ref[...]
    n = cmp_r.shape[0]

    bidx = jax.lax.broadcasted_iota(jnp.int32, (B, n), 0)
    mh = (bidx == segh_row).astype(jnp.float32)
    mp = (bidx == segp_row).astype(jnp.float32)

    counts_h = jnp.sum(mh, axis=1, keepdims=True)
    counts_p = jnp.sum(mp, axis=1, keepdims=True)
    len_h = jnp.max(counts_h)
    len_p = jnp.max(counts_p)

    segV = jnp.dot(mp, v, preferred_element_type=jnp.float32)
    php = segV / len_p
    w1 = w1_ref[...]
    u = (jnp.dot(php, w1[0:D] + w1[2 * D:3 * D], preferred_element_type=jnp.float32)
         + b1_ref[...])
    u = jnp.maximum(u, 0.0)
    cmp_pad = jnp.dot(u, w2_ref[...], preferred_element_type=jnp.float32) + b2_ref[...]

    row_sum = jnp.dot(mh, cmp_r, preferred_element_type=jnp.float32)
    segh_col = segh_col_ref[...]
    maxes = []
    for b in range(B):
        mb = jnp.where(segh_col == b, cmp_r, -jnp.inf)
        maxes.append(jnp.max(mb, axis=0, keepdims=True))
    row_max = jnp.concatenate(maxes, axis=0)

    has_pad = counts_h < len_h
    sent_max = jnp.where(has_pad, jnp.maximum(row_max, cmp_pad), row_max)
    n_pad = len_h - counts_h
    sent_mean = (row_sum + n_pad * cmp_pad) / len_h

    wc1 = wc1_ref[...]
    t = (jnp.dot(sent_max, wc1[0:D], preferred_element_type=jnp.float32)
         + jnp.dot(sent_mean, wc1[D:2 * D], preferred_element_type=jnp.float32)
         + bc1_ref[...])
    t = jnp.maximum(t, 0.0)
    logits = jnp.dot(t, wc2_ref[...], preferred_element_type=jnp.float32) + bc2_ref[...]
    logits_ref[...] = logits

    z = label_ref[...]
    lane = jax.lax.broadcasted_iota(jnp.int32, (B, 128), 1)
    term = jnp.maximum(logits, 0.0) - logits * z + jnp.log1p(jnp.exp(-jnp.abs(logits)))
    term = jnp.where(lane < C, term, 0.0)
    loss_ref[...] = jnp.sum(term, keepdims=True).reshape(1, 1) / (B * C)


def _final(cmp_r, V, segh_row, segh_col, segp_row, W1, b1, W2, b2,
           Wc1, bc1, Wc2p, bc2p, label_p):
    n = cmp_r.shape[0]
    return pl.pallas_call(
        _final_body,
        grid=(1,),
        in_specs=[_full((n, D)), _full((n, D)), _full((1, n)), _full((n, 1)),
                  _full((1, n)), _full((4 * D, D)), _full((1, D)), _full((D, D)),
                  _full((1, D)), _full((2 * D, D)), _full((1, D)), _full((D, 128)),
                  _full((1, 128)), _full((B, 128))],
        out_specs=[_full((B, 128)), _full((1, 1))],
        out_shape=[
            jax.ShapeDtypeStruct((B, 128), jnp.float32),
            jax.ShapeDtypeStruct((1, 1), jnp.float32),
        ],
    )(cmp_r, V, segh_row, segh_col, segp_row, W1, b1, W2, b2,
      Wc1, bc1, Wc2p, bc2p, label_p)


ETILE = 512


def _edge_body(xh_ref, ps_ref, pd_ref, shr_ref, hsel_ref,
               srcc_ref, dstc_ref, dstr_ref, out_ref, acc_ref):
    t = pl.program_id(0)
    nt = pl.num_programs(0)
    n = xh_ref.shape[0]

    src_col = srcc_ref[...].reshape(ETILE, 1)
    dst_col = dstc_ref[...].reshape(ETILE, 1)
    dst_row = dstr_ref[...].reshape(1, ETILE)
    lane_e = jax.lax.broadcasted_iota(jnp.int32, (ETILE, n), 1)
    oh_src = (lane_e == src_col).astype(jnp.bfloat16)
    oh_dst = (lane_e == dst_col).astype(jnp.bfloat16)

    a = (jnp.dot(oh_src, ps_ref[...], preferred_element_type=jnp.float32)
         + jnp.dot(oh_dst, pd_ref[...], preferred_element_type=jnp.float32))
    a = jnp.where(a > 0, a, NEG_SLOPE * a)
    lane = jax.lax.broadcasted_iota(jnp.int32, (ETILE, 128), 1)
    ex = jnp.where(lane < H, jnp.exp(a - shr_ref[...]), 0.0)
    ex_full = jnp.dot(ex.astype(jnp.bfloat16), hsel_ref[...],
                      preferred_element_type=jnp.float32)
    g = jnp.dot(oh_src, xh_ref[...],
                preferred_element_type=jnp.float32)
    msg = (g * ex_full).astype(jnp.bfloat16)
    pay = jnp.concatenate([msg, ex.astype(jnp.bfloat16)], axis=1)
    oh_dst_t = (jax.lax.broadcasted_iota(jnp.int32, (n, ETILE), 0)
                == dst_row).astype(jnp.bfloat16)
    contrib = jnp.dot(oh_dst_t, pay, preferred_element_type=jnp.float32)
    acc = jnp.where(t == 0, contrib, acc_ref[...] + contrib)
    acc_ref[...] = acc

    @pl.when(t == nt - 1)
    def _finish():
        lane_n = jax.lax.broadcasted_iota(jnp.int32, (n, 128), 1)
        a_self = ps_ref[...].astype(jnp.float32) + pd_ref[...].astype(jnp.float32)
        a_self = jnp.where(a_self > 0, a_self, NEG_SLOPE * a_self)
        ex_self = jnp.where(lane_n < H, jnp.exp(a_self - shr_ref[...]), 0.0)
        exs_full = jnp.dot(ex_self.astype(jnp.bfloat16), hsel_ref[...],
                           preferred_element_type=jnp.float32)
        m_self = exs_full * xh_ref[...].astype(jnp.float32)
        dfull = jnp.dot((acc[:, D:] + ex_self).astype(jnp.bfloat16),
                        hsel_ref[...],
                        preferred_element_type=jnp.float32)
        out_ref[...] = (acc[:, :D] + m_self) / (dfull + 1e-16)


def _gat_edge_pallas(xh_bf, pack_s, pack_d, shift_row, hsel, src_col, dst_col,
                     dst_row):
    n = xh_bf.shape[0]
    ntiles = src_col.shape[0]
    return pl.pallas_call(
        _edge_body,
        grid=(ntiles,),
        in_specs=[
            _full((n, D)), _full((n, 128)), _full((n, 128)),
            _full((1, 128)), _full((128, D)),
            pl.BlockSpec((1, ETILE, 1), lambda t: (t, 0, 0)),
            pl.BlockSpec((1, ETILE, 1), lambda t: (t, 0, 0)),
            pl.BlockSpec((1, 1, ETILE), lambda t: (t, 0, 0)),
        ],
        out_specs=_full((n, D)),
        out_shape=jax.ShapeDtypeStruct((n, D), jnp.float32),
        scratch_shapes=[
            pltpu.VMEM((n, D + 128), jnp.float32),
        ],
    )(xh_bf, pack_s, pack_d, shift_row, hsel, src_col, dst_col, dst_row)


def _gat_edge(xh, asd, src_col, dst_col, dst_row, hsel, lane4):
    n = xh.shape[0]
    a_max = jnp.max(asd[:, 0:2 * H], axis=0)
    c = a_max[0:H] + a_max[H:2 * H]
    c = jnp.where(c > 0, c, NEG_SLOPE * c)
    shift_row = jnp.zeros((1, 128), jnp.float32).at[0, 0:H].set(c)
    pack_s = jnp.where(lane4, asd, 0.0).astype(jnp.bfloat16)
    pack_d = jnp.where(lane4, jnp.roll(asd, -H, axis=1), 0.0).astype(jnp.bfloat16)
    xh_bf = xh.astype(jnp.bfloat16)
    return _gat_edge_pallas(xh_bf, pack_s, pack_d, shift_row, hsel,
                            src_col, dst_col, dst_row)


def _encoder(x, src_col, dst_col, dst_row, hsel, lane4, bias0, b_gat_row,
             W_gat, A_pack):
    bias = bias0
    for _ in range(NUM_LAYERS):
        xh, asd = _gat_node(x, bias, W_gat, A_pack)
        x = _gat_edge(xh, asd, src_col, dst_col, dst_row, hsel, lane4)
        bias = b_gat_row
    return x


def kernel(x_p, x_h, edge_index_p, edge_index_h, x_p_batch, x_h_batch, label,
           emb, W_gat, att_src, att_dst, b_gat, Wq, Wk, Wv, W1, b1, W2, b2,
           Wc1, bc1, Wc2, bc2):
    n = x_p.shape[0]

    rows = jnp.arange(D)
    head = rows // OUT
    A_pack = jnp.zeros((D, 128), jnp.float32)
    A_pack = A_pack.at[rows, head].set(att_src.reshape(-1))
    A_pack = A_pack.at[rows, head + H].set(att_dst.reshape(-1))
    zero_row = jnp.zeros((1, D), jnp.float32)
    b_gat_row = b_gat.reshape(1, D)
    b1_row = b1.reshape(1, D)
    b2_row = b2.reshape(1, D)
    bc1_row = bc1.reshape(1, D)
    Wc2p = jnp.zeros((D, 128), jnp.float32).at[:, 0:C].set(Wc2)
    bc2p = jnp.zeros((1, 128), jnp.float32).at[0, 0:C].set(bc2)
    label_p = jnp.zeros((B, 128), jnp.float32).at[:, 0:C].set(label.reshape(-1, C))
    segh_row = x_h_batch.reshape(1, n).astype(jnp.int32)
    segh_col = x_h_batch.reshape(n, 1).astype(jnp.int32)
    segp_row = x_p_batch.reshape(1, n).astype(jnp.int32)

    w_cat = jnp.take(emb, jnp.concatenate([x_p, x_h]), axis=0)
    src = jnp.concatenate([edge_index_p[0].astype(jnp.int32),
                           edge_index_h[0].astype(jnp.int32) + n])
    dst = jnp.concatenate([edge_index_p[1].astype(jnp.int32),
                           edge_index_h[1].astype(jnp.int32) + n])
    ntiles = src.shape[0] // ETILE
    src_col = src.reshape(ntiles, ETILE, 1)
    dst_col = dst.reshape(ntiles, ETILE, 1)
    dst_row = dst.reshape(ntiles, 1, ETILE)
    lane128 = jnp.arange(128)[None, :]
    lane4 = lane128 < H
    hsel = (jnp.arange(D)[None, :] // OUT == jnp.arange(128)[:, None]
            ).astype(jnp.bfloat16)
    x_enc = _encoder(w_cat, src_col, dst_col, dst_row, hsel, lane4,
                     zero_row, b_gat_row, W_gat, A_pack)
    p_raw = x_enc[:n]
    h_raw = x_enc[n:]

    Q, K, V = _qkv(h_raw, p_raw, b_gat_row, Wq, Wk, Wv)
    p_hat = _attention(Q, K, V, segh_col, segp_row)
    cmp_r = _ffn(p_hat, h_raw, b_gat_row, W1, b1_row, W2, b2_row)
    logits_p, loss = _final(cmp_r, V, segh_row, segh_col, segp_row,
                            W1, b1_row, W2, b2_row, Wc1, bc1_row, Wc2p, bc2p,
                            label_p)
    logits = logits_p[:, 0:C]
    return (loss.reshape(()), logits)
